# baseline (speedup 1.0000x reference)
import functools

import jax
import jax.numpy as jnp
from jax import lax
from jax.experimental import pallas as pl
from jax.experimental.pallas import tpu as pltpu
from jax.experimental.pallas import tpu_sc as plsc

_N = 10000
_E = 320000
_F = 128
_H = 256
_OUT = 128
_G = 128
_NEG = 0.01

_NC = 2
_NS = 16
_NW = _NC * _NS
_CH = 128
_IB = 40
_K1 = 80
_K2 = 158
_NP = 10112
_ZPW = _NP // _NS
_RPAD = 10240
_RPW = _RPAD // _NW
_TRASH = _N


def _leaky(x):
    return jnp.where(x >= 0, x, _NEG * x)


def _sc_mesh():
    return plsc.VectorSubcoreMesh(core_axis_name="c", subcore_axis_name="s")


def _make_agg(K, per_core_rows):
    @functools.partial(
        pl.kernel,
        out_type=jax.ShapeDtypeStruct((_NC, _NP, _CH), jnp.float32),
        mesh=_sc_mesh(),
        scratch_types=[
            pltpu.VMEM((_CH,), jnp.int32),
            pltpu.VMEM((_CH,), jnp.int32),
            pltpu.VMEM((_CH, _CH), jnp.float32),
            pltpu.VMEM_SHARED((_NP, _CH), jnp.float32),
            pltpu.SemaphoreType.DMA,
        ],
    )
    def agg(table, srcm, dstm, out, idx_v, dst_v, rows_v, acc, sem):
        c = lax.axis_index("c")
        s = lax.axis_index("s")

        def zrow(i, carry):
            for k in range(_CH // 16):
                rows_v[i, pl.ds(k * 16, 16)] = jnp.zeros((16,), jnp.float32)
            return carry

        lax.fori_loop(0, _CH, zrow, 0)
        zbase = s * _ZPW
        off = 0
        while off < _ZPW:
            ch = min(_CH, _ZPW - off)
            pltpu.sync_copy(rows_v.at[pl.ds(0, ch)],
                            acc.at[pl.ds(zbase + off, ch)])
            off += ch
        plsc.subcore_barrier()

        rbase = per_core_rows(c, s) * K

        def body(j, carry):
            row = rbase + j
            pltpu.sync_copy(srcm.at[row], idx_v)
            pltpu.async_copy(table.at[idx_v], rows_v, sem).wait()
            pltpu.sync_copy(dstm.at[row], dst_v)
            pltpu.sync_copy(rows_v, acc.at[dst_v], add=True)
            return carry

        lax.fori_loop(0, K, body, 0)
        plsc.subcore_barrier()
        pltpu.sync_copy(acc.at[pl.ds(zbase, _ZPW)],
                        out.at[c, pl.ds(zbase, _ZPW)])

    return agg


_HS = _H + 16


@functools.partial(
    pl.kernel,
    out_type=(
        jax.ShapeDtypeStruct((_NW, _G * _HS), jnp.float32),
        jax.ShapeDtypeStruct((_NW, _G * _H), jnp.float32),
    ),
    mesh=_sc_mesh(),
    scratch_types=[
        pltpu.VMEM((_RPW + 16,), jnp.int32),
        pltpu.VMEM((32 * _H,), jnp.float32),
        pltpu.VMEM(((_G + 1) * _HS,), jnp.float32),
        pltpu.VMEM(((_G + 1) * _H,), jnp.float32),
    ],
)
def _pool(h2f, batchp, sums, maxs, batch_v, buf, sacc, macc):
    c = lax.axis_index("c")
    s = lax.axis_index("s")
    wid = s * _NC + c
    base = wid * _RPW

    zero16 = jnp.zeros((16,), jnp.float32)
    ninf16 = jnp.full((16,), -3.4e38, jnp.float32)
    one16 = jnp.full((16,), 1.0, jnp.float32)

    def inits(i, carry):
        sacc[pl.ds(i * 16, 16)] = zero16
        return carry

    def initm(i, carry):
        macc[pl.ds(i * 16, 16)] = ninf16
        return carry

    lax.fori_loop(0, (_G + 1) * _HS // 16, inits, 0)
    lax.fori_loop(0, (_G + 1) * _H // 16, initm, 0)

    pltpu.sync_copy(batchp.at[pl.ds(base, _RPW)], batch_v.at[pl.ds(0, _RPW)])

    def chunk(t, carry):
        pltpu.sync_copy(h2f.at[pl.ds((base + t * 32) * _H, 32 * _H)], buf)

        def row(i, carry2):
            r = t * 32 + i
            g = batch_v[pl.ds(r, 16)][0]
            soff = g * _HS
            moff = g * _H
            for k in range(_H // 16):
                v = buf[pl.ds(i * _H + k * 16, 16)]
                sacc[pl.ds(soff + k * 16, 16)] = sacc[pl.ds(soff + k * 16, 16)] + v
                m = macc[pl.ds(moff + k * 16, 16)]
                macc[pl.ds(moff + k * 16, 16)] = jnp.maximum(m, v)
            sacc[pl.ds(soff + _H, 16)] = sacc[pl.ds(soff + _H, 16)] + one16
            return carry2

        lax.fori_loop(0, 32, row, 0)
        return carry

    lax.fori_loop(0, _RPW // 32, chunk, 0)

    pltpu.sync_copy(sacc.at[pl.ds(0, _G * _HS)], sums.at[wid])
    pltpu.sync_copy(macc.at[pl.ds(0, _G * _H)], maxs.at[wid])


def _tc_layer1(x, parts, W0, b0):
    def body(x_ref, p_ref, w_ref, b_ref, o_ref):
        z = x_ref[...] + p_ref[0] + p_ref[1]
        z = jnp.dot(z, w_ref[...], preferred_element_type=jnp.float32)
        z = z + b_ref[...]
        o_ref[...] = _leaky(_leaky(z))

    blk = 1024
    return pl.pallas_call(
        body,
        grid=(_RPAD // blk,),
        in_specs=[
            pl.BlockSpec((blk, _F), lambda i: (i, 0)),
            pl.BlockSpec((_NC, blk, _F), lambda i: (0, i, 0)),
            pl.BlockSpec((_F, _H), lambda i: (0, 0)),
            pl.BlockSpec((1, _H), lambda i: (0, 0)),
        ],
        out_specs=pl.BlockSpec((blk, _H), lambda i: (i, 0)),
        out_shape=jax.ShapeDtypeStruct((_RPAD, _H), jnp.float32),
    )(x, parts, W0, b0)


def _tc_layer2(h1, parts, W1, b1):
    def body(h_ref, p_ref, w_ref, b_ref, o_ref):
        h = h_ref[...]
        za = h[:, :_F] + p_ref[0]
        zb = h[:, _F:] + p_ref[1]
        w = w_ref[...]
        z = jnp.dot(za, w[:_F, :], preferred_element_type=jnp.float32)
        z = z + jnp.dot(zb, w[_F:, :], preferred_element_type=jnp.float32)
        z = z + b_ref[...]
        o_ref[...] = _leaky(_leaky(z))

    blk = 1024
    return pl.pallas_call(
        body,
        grid=(_RPAD // blk,),
        in_specs=[
            pl.BlockSpec((blk, _H), lambda i: (i, 0)),
            pl.BlockSpec((_NC, blk, _F), lambda i: (0, i, 0)),
            pl.BlockSpec((_H, _H), lambda i: (0, 0)),
            pl.BlockSpec((1, _H), lambda i: (0, 0)),
        ],
        out_specs=pl.BlockSpec((blk, _H), lambda i: (i, 0)),
        out_shape=jax.ShapeDtypeStruct((_RPAD, _H), jnp.float32),
    )(h1, parts, W1, b1)


def _tc_pool_combine(sums, maxs, Wp, bp):
    def body(s_ref, m_ref, w_ref, b_ref, o_ref):
        sall = jnp.sum(s_ref[...], axis=0)
        ssum = sall[:, :_H]
        cnt = sall[:, _H]
        mx = jnp.max(m_ref[...], axis=0)
        mean = ssum / jnp.maximum(cnt, 1.0)[:, None]
        mx = jnp.where(cnt[:, None] > 0, mx, 0.0)
        g = jnp.concatenate([mean, ssum, mx], axis=-1)
        z = jnp.dot(g, w_ref[...], preferred_element_type=jnp.float32)
        o_ref[...] = _leaky(z + b_ref[...])

    return pl.pallas_call(
        body,
        grid=(1,),
        in_specs=[
            pl.BlockSpec((_NW, _G, _HS), lambda i: (0, 0, 0)),
            pl.BlockSpec((_NW, _G, _H), lambda i: (0, 0, 0)),
            pl.BlockSpec((3 * _H, _OUT), lambda i: (0, 0)),
            pl.BlockSpec((1, _OUT), lambda i: (0, 0)),
        ],
        out_specs=pl.BlockSpec((_G, _OUT), lambda i: (0, 0)),
        out_shape=jax.ShapeDtypeStruct((_G, _OUT), jnp.float32),
    )(sums, maxs, Wp, bp)


def kernel(x, edge_index, batch, W0, b0, W1, b1, Wp, bp):
    src = edge_index[0].astype(jnp.int32)
    dst = edge_index[1].astype(jnp.int32)

    pad1 = _NW * _K1 * _CH - _E
    s1 = jnp.concatenate([2 * src, jnp.zeros((pad1,), jnp.int32)])
    s1 = s1.reshape(_NW, -1) + (jnp.arange(_NW, dtype=jnp.int32) % _NC)[:, None]
    src1 = s1.reshape(-1, _CH)
    dst1 = jnp.concatenate([dst, jnp.full((pad1,), _TRASH, jnp.int32)]).reshape(-1, _CH)
    pad2 = _NS * _K2 * _CH - _E
    s2 = jnp.concatenate([2 * src, jnp.zeros((pad2,), jnp.int32)])
    src2 = jnp.concatenate([s2, s2 + 1]).reshape(-1, _CH)
    d2 = jnp.concatenate([dst, jnp.full((pad2,), _TRASH, jnp.int32)])
    dst2 = jnp.concatenate([d2, d2]).reshape(-1, _CH)

    table1 = jnp.repeat(x, 2, axis=0)
    agg1 = _make_agg(_K1, lambda c, s: s * _NC + c)(table1, src1, dst1)
    h1 = _tc_layer1(x, agg1, W0, b0.reshape(1, _H))

    table2 = h1.reshape(2 * _RPAD, _F)
    agg2 = _make_agg(_K2, lambda c, s: c * _NS + s)(table2, src2, dst2)
    h2 = _tc_layer2(h1, agg2, W1, b1.reshape(1, _H))

    batchp = jnp.concatenate(
        [batch.astype(jnp.int32), jnp.full((_RPAD - _N,), _G, jnp.int32)])
    sums, maxs = _pool(h2.reshape(-1), batchp)

    return _tc_pool_combine(sums.reshape(_NW, _G, _HS),
                            maxs.reshape(_NW, _G, _H),
                            Wp, bp.reshape(1, _OUT))


# speedup vs baseline: 1.0794x; 1.0794x over previous
import functools

import jax
import jax.numpy as jnp
from jax import lax
from jax.experimental import pallas as pl
from jax.experimental.pallas import tpu as pltpu
from jax.experimental.pallas import tpu_sc as plsc

_N = 10000
_E = 320000
_F = 128
_H = 256
_OUT = 128
_G = 128
_NEG = 0.01

_NC = 2
_NS = 16
_NW = _NC * _NS
_CH = 128
_IB = 40
_K1 = 80
_K2 = 158
_NP = 10112
_ZPW = _NP // _NS
_RPAD = 10240
_RPW = _RPAD // _NW
_TRASH = _N


def _leaky(x):
    return jnp.where(x >= 0, x, _NEG * x)


def _sc_mesh():
    return plsc.VectorSubcoreMesh(core_axis_name="c", subcore_axis_name="s")


def _make_agg(K, add_core):
    @functools.partial(
        pl.kernel,
        out_type=jax.ShapeDtypeStruct((_NC, _NP, _CH), jnp.float32),
        mesh=_sc_mesh(),
        scratch_types=[
            pltpu.VMEM((_CH,), jnp.int32),
            pltpu.VMEM((_CH,), jnp.int32),
            pltpu.VMEM((_CH, _CH), jnp.float32),
            pltpu.VMEM_SHARED((_NP, _CH), jnp.float32),
            pltpu.SemaphoreType.DMA,
        ],
    )
    def agg(table, srcm, dstm, out, idx_v, dst_v, rows_v, acc, sem):
        c = lax.axis_index("c")
        s = lax.axis_index("s")

        def zrow(i, carry):
            for k in range(_CH // 16):
                rows_v[i, pl.ds(k * 16, 16)] = jnp.zeros((16,), jnp.float32)
            return carry

        lax.fori_loop(0, _CH, zrow, 0)
        zbase = s * _ZPW
        off = 0
        while off < _ZPW:
            ch = min(_CH, _ZPW - off)
            pltpu.sync_copy(rows_v.at[pl.ds(0, ch)],
                            acc.at[pl.ds(zbase + off, ch)])
            off += ch
        plsc.subcore_barrier()

        if add_core:
            rbase = s * K
        else:
            rbase = (s * _NC + c) * K

        cvec = jnp.full((16,), c, jnp.int32)

        def body(j, carry):
            row = rbase + j
            pltpu.sync_copy(srcm.at[row], idx_v)
            if add_core:
                for k in range(_CH // 16):
                    idx_v[pl.ds(k * 16, 16)] = idx_v[pl.ds(k * 16, 16)] + cvec
            g = pltpu.async_copy(table.at[idx_v], rows_v, sem)
            pltpu.sync_copy(dstm.at[row], dst_v)
            g.wait()
            pltpu.sync_copy(rows_v, acc.at[dst_v], add=True)
            return carry

        lax.fori_loop(0, K, body, 0)
        plsc.subcore_barrier()
        pltpu.sync_copy(acc.at[pl.ds(zbase, _ZPW)],
                        out.at[c, pl.ds(zbase, _ZPW)])

    return agg


_HS = _H + 16


@functools.partial(
    pl.kernel,
    out_type=(
        jax.ShapeDtypeStruct((_NW, _G * _HS), jnp.float32),
        jax.ShapeDtypeStruct((_NW, _G * _H), jnp.float32),
    ),
    mesh=_sc_mesh(),
    scratch_types=[
        pltpu.VMEM((_RPW + 16,), jnp.int32),
        pltpu.VMEM((32 * _H,), jnp.float32),
        pltpu.VMEM(((_G + 1) * _HS,), jnp.float32),
        pltpu.VMEM(((_G + 1) * _H,), jnp.float32),
    ],
)
def _pool(h2f, batchp, sums, maxs, batch_v, buf, sacc, macc):
    c = lax.axis_index("c")
    s = lax.axis_index("s")
    wid = s * _NC + c
    base = wid * _RPW

    zero16 = jnp.zeros((16,), jnp.float32)
    ninf16 = jnp.full((16,), -3.4e38, jnp.float32)
    one16 = jnp.full((16,), 1.0, jnp.float32)

    def inits(i, carry):
        sacc[pl.ds(i * 16, 16)] = zero16
        return carry

    def initm(i, carry):
        macc[pl.ds(i * 16, 16)] = ninf16
        return carry

    lax.fori_loop(0, (_G + 1) * _HS // 16, inits, 0)
    lax.fori_loop(0, (_G + 1) * _H // 16, initm, 0)

    pltpu.sync_copy(batchp.at[pl.ds(base, _RPW)], batch_v.at[pl.ds(0, _RPW)])

    def chunk(t, carry):
        pltpu.sync_copy(h2f.at[pl.ds((base + t * 32) * _H, 32 * _H)], buf)

        def row(i, carry2):
            r = t * 32 + i
            g = batch_v[pl.ds(r, 16)][0]
            soff = g * _HS
            moff = g * _H
            for k in range(_H // 16):
                v = buf[pl.ds(i * _H + k * 16, 16)]
                sacc[pl.ds(soff + k * 16, 16)] = sacc[pl.ds(soff + k * 16, 16)] + v
                m = macc[pl.ds(moff + k * 16, 16)]
                macc[pl.ds(moff + k * 16, 16)] = jnp.maximum(m, v)
            sacc[pl.ds(soff + _H, 16)] = sacc[pl.ds(soff + _H, 16)] + one16
            return carry2

        lax.fori_loop(0, 32, row, 0)
        return carry

    lax.fori_loop(0, _RPW // 32, chunk, 0)

    pltpu.sync_copy(sacc.at[pl.ds(0, _G * _HS)], sums.at[wid])
    pltpu.sync_copy(macc.at[pl.ds(0, _G * _H)], maxs.at[wid])


def _tc_layer1(x, parts, W0, b0):
    def body(x_ref, p_ref, w_ref, b_ref, o_ref):
        z = x_ref[...] + p_ref[0] + p_ref[1]
        z = jnp.dot(z, w_ref[...], preferred_element_type=jnp.float32)
        z = z + b_ref[...]
        o_ref[...] = _leaky(_leaky(z))

    blk = 1024
    return pl.pallas_call(
        body,
        grid=(_RPAD // blk,),
        in_specs=[
            pl.BlockSpec((blk, _F), lambda i: (i, 0)),
            pl.BlockSpec((_NC, blk, _F), lambda i: (0, i, 0)),
            pl.BlockSpec((_F, _H), lambda i: (0, 0)),
            pl.BlockSpec((1, _H), lambda i: (0, 0)),
        ],
        out_specs=pl.BlockSpec((blk, _H), lambda i: (i, 0)),
        out_shape=jax.ShapeDtypeStruct((_RPAD, _H), jnp.float32),
    )(x, parts, W0, b0)


def _tc_layer2(h1, parts, W1, b1):
    def body(h_ref, p_ref, w_ref, b_ref, o_ref):
        h = h_ref[...]
        za = h[:, :_F] + p_ref[0]
        zb = h[:, _F:] + p_ref[1]
        w = w_ref[...]
        z = jnp.dot(za, w[:_F, :], preferred_element_type=jnp.float32)
        z = z + jnp.dot(zb, w[_F:, :], preferred_element_type=jnp.float32)
        z = z + b_ref[...]
        o_ref[...] = _leaky(_leaky(z))

    blk = 1024
    return pl.pallas_call(
        body,
        grid=(_RPAD // blk,),
        in_specs=[
            pl.BlockSpec((blk, _H), lambda i: (i, 0)),
            pl.BlockSpec((_NC, blk, _F), lambda i: (0, i, 0)),
            pl.BlockSpec((_H, _H), lambda i: (0, 0)),
            pl.BlockSpec((1, _H), lambda i: (0, 0)),
        ],
        out_specs=pl.BlockSpec((blk, _H), lambda i: (i, 0)),
        out_shape=jax.ShapeDtypeStruct((_RPAD, _H), jnp.float32),
    )(h1, parts, W1, b1)


def _tc_pool_combine(sums, maxs, Wp, bp):
    def body(s_ref, m_ref, w_ref, b_ref, o_ref):
        sall = jnp.sum(s_ref[...], axis=0)
        ssum = sall[:, :_H]
        cnt = sall[:, _H]
        mx = jnp.max(m_ref[...], axis=0)
        mean = ssum / jnp.maximum(cnt, 1.0)[:, None]
        mx = jnp.where(cnt[:, None] > 0, mx, 0.0)
        g = jnp.concatenate([mean, ssum, mx], axis=-1)
        z = jnp.dot(g, w_ref[...], preferred_element_type=jnp.float32)
        o_ref[...] = _leaky(z + b_ref[...])

    return pl.pallas_call(
        body,
        grid=(1,),
        in_specs=[
            pl.BlockSpec((_NW, _G, _HS), lambda i: (0, 0, 0)),
            pl.BlockSpec((_NW, _G, _H), lambda i: (0, 0, 0)),
            pl.BlockSpec((3 * _H, _OUT), lambda i: (0, 0)),
            pl.BlockSpec((1, _OUT), lambda i: (0, 0)),
        ],
        out_specs=pl.BlockSpec((_G, _OUT), lambda i: (0, 0)),
        out_shape=jax.ShapeDtypeStruct((_G, _OUT), jnp.float32),
    )(sums, maxs, Wp, bp)


def kernel(x, edge_index, batch, W0, b0, W1, b1, Wp, bp):
    src = edge_index[0].astype(jnp.int32)
    dst = edge_index[1].astype(jnp.int32)

    pad1 = _NW * _K1 * _CH - _E
    src1 = jnp.concatenate([src, jnp.zeros((pad1,), jnp.int32)]).reshape(-1, _CH)
    dst1 = jnp.concatenate([dst, jnp.full((pad1,), _TRASH, jnp.int32)]).reshape(-1, _CH)
    pad2 = _NS * _K2 * _CH - _E
    src2 = jnp.concatenate([2 * src, jnp.zeros((pad2,), jnp.int32)]).reshape(-1, _CH)
    dst2 = jnp.concatenate([dst, jnp.full((pad2,), _TRASH, jnp.int32)]).reshape(-1, _CH)

    agg1 = _make_agg(_K1, add_core=False)(x, src1, dst1)
    h1 = _tc_layer1(x, agg1, W0, b0.reshape(1, _H))

    table2 = h1.reshape(2 * _RPAD, _F)
    agg2 = _make_agg(_K2, add_core=True)(table2, src2, dst2)
    h2 = _tc_layer2(h1, agg2, W1, b1.reshape(1, _H))

    batchp = jnp.concatenate(
        [batch.astype(jnp.int32), jnp.full((_RPAD - _N,), _G, jnp.int32)])
    sums, maxs = _pool(h2.reshape(-1), batchp)

    return _tc_pool_combine(sums.reshape(_NW, _G, _HS),
                            maxs.reshape(_NW, _G, _H),
                            Wp, bp.reshape(1, _OUT))


# speedup vs baseline: 1.2662x; 1.1731x over previous
import functools

import jax
import jax.numpy as jnp
from jax import lax
from jax.experimental import pallas as pl
from jax.experimental.pallas import tpu as pltpu
from jax.experimental.pallas import tpu_sc as plsc

_N = 10000
_E = 320000
_F = 128
_H = 256
_OUT = 128
_G = 128
_NEG = 0.01

_NC = 2
_NS = 16
_NW = _NC * _NS
_CH = 128
_IB = 40
_K1 = 79
_K2 = 157
_NP = 10112
_ZPW = _NP // _NS
_RPAD = 10240
_RPW = _RPAD // _NW
_TRASH = _N


def _leaky(x):
    return jnp.where(x >= 0, x, _NEG * x)


def _sc_mesh():
    return plsc.VectorSubcoreMesh(core_axis_name="c", subcore_axis_name="s")


def _make_agg(K, add_core):
    @functools.partial(
        pl.kernel,
        out_type=jax.ShapeDtypeStruct((_NC, _NP, _CH), jnp.float32),
        mesh=_sc_mesh(),
        scratch_types=[
            pltpu.VMEM((_CH,), jnp.int32),
            pltpu.VMEM((_CH,), jnp.int32),
            pltpu.VMEM((_CH, _CH), jnp.float32),
            pltpu.VMEM_SHARED((_NP, _CH), jnp.float32),
            pltpu.SemaphoreType.DMA,
        ],
    )
    def agg(table, srcm, dstm, out, idx_v, dst_v, rows_v, acc, sem):
        c = lax.axis_index("c")
        s = lax.axis_index("s")

        def zrow(i, carry):
            for k in range(_CH // 16):
                rows_v[i, pl.ds(k * 16, 16)] = jnp.zeros((16,), jnp.float32)
            return carry

        lax.fori_loop(0, _CH, zrow, 0)
        zbase = s * _ZPW
        off = 0
        while off < _ZPW:
            ch = min(_CH, _ZPW - off)
            pltpu.sync_copy(rows_v.at[pl.ds(0, ch)],
                            acc.at[pl.ds(zbase + off, ch)])
            off += ch
        plsc.subcore_barrier()

        if add_core:
            rbase = s * K
        else:
            rbase = (s * _NC + c) * K

        cvec = jnp.full((16,), c, jnp.int32)

        def body(j, carry):
            row = rbase + j
            pltpu.sync_copy(srcm.at[row], idx_v)
            if add_core:
                for k in range(_CH // 16):
                    idx_v[pl.ds(k * 16, 16)] = idx_v[pl.ds(k * 16, 16)] + cvec
            pltpu.async_copy(table.at[idx_v], rows_v, sem).wait()
            pltpu.sync_copy(dstm.at[row], dst_v)
            pltpu.sync_copy(rows_v, acc.at[dst_v], add=True)
            return carry

        lax.fori_loop(0, K, body, 0)
        plsc.subcore_barrier()
        pltpu.sync_copy(acc.at[pl.ds(zbase, _ZPW)],
                        out.at[c, pl.ds(zbase, _ZPW)])

    return agg


_HS = _H + 16


@functools.partial(
    pl.kernel,
    out_type=(
        jax.ShapeDtypeStruct((_NW, _G * _HS), jnp.float32),
        jax.ShapeDtypeStruct((_NW, _G * _H), jnp.float32),
    ),
    mesh=_sc_mesh(),
    scratch_types=[
        pltpu.VMEM((_RPW + 16,), jnp.int32),
        pltpu.VMEM((32 * _H,), jnp.float32),
        pltpu.VMEM(((_G + 1) * _HS,), jnp.float32),
        pltpu.VMEM(((_G + 1) * _H,), jnp.float32),
    ],
)
def _pool(h2f, batchp, sums, maxs, batch_v, buf, sacc, macc):
    c = lax.axis_index("c")
    s = lax.axis_index("s")
    wid = s * _NC + c
    base = wid * _RPW

    zero16 = jnp.zeros((16,), jnp.float32)
    ninf16 = jnp.full((16,), -3.4e38, jnp.float32)
    one16 = jnp.full((16,), 1.0, jnp.float32)

    def inits(i, carry):
        sacc[pl.ds(i * 16, 16)] = zero16
        return carry

    def initm(i, carry):
        macc[pl.ds(i * 16, 16)] = ninf16
        return carry

    lax.fori_loop(0, (_G + 1) * _HS // 16, inits, 0)
    lax.fori_loop(0, (_G + 1) * _H // 16, initm, 0)

    pltpu.sync_copy(batchp.at[pl.ds(base, _RPW)], batch_v.at[pl.ds(0, _RPW)])

    def chunk(t, carry):
        pltpu.sync_copy(h2f.at[pl.ds((base + t * 32) * _H, 32 * _H)], buf)

        def row(i, carry2):
            r = t * 32 + i
            g = batch_v[pl.ds(r, 16)][0]
            soff = g * _HS
            moff = g * _H
            for k in range(_H // 16):
                v = buf[pl.ds(i * _H + k * 16, 16)]
                sacc[pl.ds(soff + k * 16, 16)] = sacc[pl.ds(soff + k * 16, 16)] + v
                m = macc[pl.ds(moff + k * 16, 16)]
                macc[pl.ds(moff + k * 16, 16)] = jnp.maximum(m, v)
            sacc[pl.ds(soff + _H, 16)] = sacc[pl.ds(soff + _H, 16)] + one16
            return carry2

        lax.fori_loop(0, 32, row, 0)
        return carry

    lax.fori_loop(0, _RPW // 32, chunk, 0)

    pltpu.sync_copy(sacc.at[pl.ds(0, _G * _HS)], sums.at[wid])
    pltpu.sync_copy(macc.at[pl.ds(0, _G * _H)], maxs.at[wid])


def _tc_layer1(x, parts, W0, b0):
    def body(x_ref, p_ref, w_ref, b_ref, o_ref):
        z = x_ref[...] + p_ref[0] + p_ref[1]
        z = jnp.dot(z, w_ref[...], preferred_element_type=jnp.float32)
        z = z + b_ref[...]
        o_ref[...] = _leaky(_leaky(z))

    blk = 1024
    return pl.pallas_call(
        body,
        grid=(_RPAD // blk,),
        in_specs=[
            pl.BlockSpec((blk, _F), lambda i: (i, 0)),
            pl.BlockSpec((_NC, blk, _F), lambda i: (0, i, 0)),
            pl.BlockSpec((_F, _H), lambda i: (0, 0)),
            pl.BlockSpec((1, _H), lambda i: (0, 0)),
        ],
        out_specs=pl.BlockSpec((blk, _H), lambda i: (i, 0)),
        out_shape=jax.ShapeDtypeStruct((_RPAD, _H), jnp.float32),
    )(x, parts, W0, b0)


def _tc_layer2(h1, parts, W1, b1):
    def body(h_ref, p_ref, w_ref, b_ref, o_ref):
        h = h_ref[...]
        za = h[:, :_F] + p_ref[0]
        zb = h[:, _F:] + p_ref[1]
        w = w_ref[...]
        z = jnp.dot(za, w[:_F, :], preferred_element_type=jnp.float32)
        z = z + jnp.dot(zb, w[_F:, :], preferred_element_type=jnp.float32)
        z = z + b_ref[...]
        o_ref[...] = _leaky(_leaky(z))

    blk = 1024
    return pl.pallas_call(
        body,
        grid=(_RPAD // blk,),
        in_specs=[
            pl.BlockSpec((blk, _H), lambda i: (i, 0)),
            pl.BlockSpec((_NC, blk, _F), lambda i: (0, i, 0)),
            pl.BlockSpec((_H, _H), lambda i: (0, 0)),
            pl.BlockSpec((1, _H), lambda i: (0, 0)),
        ],
        out_specs=pl.BlockSpec((blk, _H), lambda i: (i, 0)),
        out_shape=jax.ShapeDtypeStruct((_RPAD, _H), jnp.float32),
    )(h1, parts, W1, b1)


def _tc_pool_combine(sums, maxs, Wp, bp):
    def body(s_ref, m_ref, w_ref, b_ref, o_ref):
        sall = jnp.sum(s_ref[...], axis=0)
        ssum = sall[:, :_H]
        cnt = sall[:, _H]
        mx = jnp.max(m_ref[...], axis=0)
        mean = ssum / jnp.maximum(cnt, 1.0)[:, None]
        mx = jnp.where(cnt[:, None] > 0, mx, 0.0)
        g = jnp.concatenate([mean, ssum, mx], axis=-1)
        z = jnp.dot(g, w_ref[...], preferred_element_type=jnp.float32)
        o_ref[...] = _leaky(z + b_ref[...])

    return pl.pallas_call(
        body,
        grid=(1,),
        in_specs=[
            pl.BlockSpec((_NW, _G, _HS), lambda i: (0, 0, 0)),
            pl.BlockSpec((_NW, _G, _H), lambda i: (0, 0, 0)),
            pl.BlockSpec((3 * _H, _OUT), lambda i: (0, 0)),
            pl.BlockSpec((1, _OUT), lambda i: (0, 0)),
        ],
        out_specs=pl.BlockSpec((_G, _OUT), lambda i: (0, 0)),
        out_shape=jax.ShapeDtypeStruct((_G, _OUT), jnp.float32),
    )(sums, maxs, Wp, bp)


def kernel(x, edge_index, batch, W0, b0, W1, b1, Wp, bp):
    src = edge_index[0].astype(jnp.int32)
    dst = edge_index[1].astype(jnp.int32)

    pad1 = _NW * _K1 * _CH - _E
    src1 = jnp.concatenate([src, jnp.zeros((pad1,), jnp.int32)]).reshape(-1, _CH)
    dst1 = jnp.concatenate([dst, jnp.full((pad1,), _TRASH, jnp.int32)]).reshape(-1, _CH)
    pad2 = _NS * _K2 * _CH - _E
    src2 = jnp.concatenate([2 * src, jnp.zeros((pad2,), jnp.int32)]).reshape(-1, _CH)
    dst2 = jnp.concatenate([dst, jnp.full((pad2,), _TRASH, jnp.int32)]).reshape(-1, _CH)

    agg1 = _make_agg(_K1, add_core=False)(x, src1, dst1)
    h1 = _tc_layer1(x, agg1, W0, b0.reshape(1, _H))

    table2 = h1.reshape(2 * _RPAD, _F)
    agg2 = _make_agg(_K2, add_core=True)(table2, src2, dst2)
    h2 = _tc_layer2(h1, agg2, W1, b1.reshape(1, _H))

    batchp = jnp.concatenate(
        [batch.astype(jnp.int32), jnp.full((_RPAD - _N,), _G, jnp.int32)])
    sums, maxs = _pool(h2.reshape(-1), batchp)

    return _tc_pool_combine(sums.reshape(_NW, _G, _HS),
                            maxs.reshape(_NW, _G, _H),
                            Wp, bp.reshape(1, _OUT))


# speedup vs baseline: 1.2829x; 1.0131x over previous
import functools

import jax
import jax.numpy as jnp
from jax import lax
from jax.experimental import pallas as pl
from jax.experimental.pallas import tpu as pltpu
from jax.experimental.pallas import tpu_sc as plsc

_N = 10000
_E = 320000
_F = 128
_H = 256
_OUT = 128
_G = 128
_NEG = 0.01

_NC = 2
_NS = 16
_NW = _NC * _NS
_CH = 128
_IB = 40
_K1 = 79
_K2 = 157
_NP = 10112
_ZPW = _NP // _NS
_RPAD = 10240
_RPW = _RPAD // _NW
_TRASH = _N


def _leaky(x):
    return jnp.where(x >= 0, x, _NEG * x)


def _sc_mesh():
    return plsc.VectorSubcoreMesh(core_axis_name="c", subcore_axis_name="s")


def _make_agg(K, add_core):
    @functools.partial(
        pl.kernel,
        out_type=jax.ShapeDtypeStruct((_NC, _NP, _CH), jnp.float32),
        mesh=_sc_mesh(),
        scratch_types=[
            pltpu.VMEM((_CH,), jnp.int32),
            pltpu.VMEM((_CH,), jnp.int32),
            pltpu.VMEM((_CH, _CH), jnp.float32),
            pltpu.VMEM_SHARED((_NP, _CH), jnp.float32),
            pltpu.SemaphoreType.DMA,
        ],
    )
    def agg(table, srcm, dstm, out, idx_v, dst_v, rows_v, acc, sem):
        c = lax.axis_index("c")
        s = lax.axis_index("s")

        def zrow(i, carry):
            for k in range(_CH // 16):
                rows_v[i, pl.ds(k * 16, 16)] = jnp.zeros((16,), jnp.float32)
            return carry

        lax.fori_loop(0, _CH, zrow, 0)
        zbase = s * _ZPW
        off = 0
        while off < _ZPW:
            ch = min(_CH, _ZPW - off)
            pltpu.sync_copy(rows_v.at[pl.ds(0, ch)],
                            acc.at[pl.ds(zbase + off, ch)])
            off += ch
        plsc.subcore_barrier()

        if add_core:
            rbase = s * K
        else:
            rbase = (s * _NC + c) * K

        cvec = jnp.full((16,), c, jnp.int32)

        def body(j, carry):
            row = rbase + j
            pltpu.sync_copy(srcm.at[row], idx_v)
            if add_core:
                for k in range(_CH // 16):
                    idx_v[pl.ds(k * 16, 16)] = idx_v[pl.ds(k * 16, 16)] + cvec
            pltpu.async_copy(table.at[idx_v], rows_v, sem).wait()
            pltpu.sync_copy(dstm.at[row], dst_v)
            pltpu.sync_copy(rows_v, acc.at[dst_v], add=True)
            return carry

        lax.fori_loop(0, K, body, 0)
        plsc.subcore_barrier()
        pltpu.sync_copy(acc.at[pl.ds(zbase, _ZPW)],
                        out.at[c, pl.ds(zbase, _ZPW)])

    return agg


_HS = _H + 16


@functools.partial(
    pl.kernel,
    out_type=(
        jax.ShapeDtypeStruct((_NW, _G * _HS), jnp.float32),
        jax.ShapeDtypeStruct((_NW, _G * _H), jnp.float32),
    ),
    mesh=_sc_mesh(),
    scratch_types=[
        pltpu.VMEM((_RPW + 16,), jnp.int32),
        pltpu.VMEM((32 * _H,), jnp.float32),
        pltpu.VMEM(((_G + 1) * _HS,), jnp.float32),
        pltpu.VMEM(((_G + 1) * _H,), jnp.float32),
    ],
)
def _pool(h2f, batchp, sums, maxs, batch_v, buf, sacc, macc):
    c = lax.axis_index("c")
    s = lax.axis_index("s")
    wid = s * _NC + c
    base = wid * _RPW

    zero16 = jnp.zeros((16,), jnp.float32)
    ninf16 = jnp.full((16,), -3.4e38, jnp.float32)
    one16 = jnp.full((16,), 1.0, jnp.float32)

    def inits(i, carry):
        for u in range(4):
            sacc[pl.ds(i * 64 + u * 16, 16)] = zero16
        return carry

    def initm(i, carry):
        for u in range(4):
            macc[pl.ds(i * 64 + u * 16, 16)] = ninf16
        return carry

    lax.fori_loop(0, (_G + 1) * _HS // 64, inits, 0)
    for r in range((_G + 1) * _HS // 64 * 64, (_G + 1) * _HS, 16):
        sacc[pl.ds(r, 16)] = zero16
    lax.fori_loop(0, (_G + 1) * _H // 64, initm, 0)

    pltpu.sync_copy(batchp.at[pl.ds(base, _RPW)], batch_v.at[pl.ds(0, _RPW)])

    def chunk(t, carry):
        pltpu.sync_copy(h2f.at[pl.ds((base + t * 32) * _H, 32 * _H)], buf)

        def row(i, carry2):
            r = t * 32 + i
            g = batch_v[pl.ds(r, 16)][0]
            soff = g * _HS
            moff = g * _H
            for k in range(_H // 16):
                v = buf[pl.ds(i * _H + k * 16, 16)]
                sacc[pl.ds(soff + k * 16, 16)] = sacc[pl.ds(soff + k * 16, 16)] + v
                m = macc[pl.ds(moff + k * 16, 16)]
                macc[pl.ds(moff + k * 16, 16)] = jnp.maximum(m, v)
            sacc[pl.ds(soff + _H, 16)] = sacc[pl.ds(soff + _H, 16)] + one16
            return carry2

        lax.fori_loop(0, 32, row, 0)
        return carry

    lax.fori_loop(0, _RPW // 32, chunk, 0)

    pltpu.sync_copy(sacc.at[pl.ds(0, _G * _HS)], sums.at[wid])
    pltpu.sync_copy(macc.at[pl.ds(0, _G * _H)], maxs.at[wid])


def _tc_layer1(x, parts, W0, b0):
    def body(x_ref, p_ref, w_ref, b_ref, o_ref):
        z = x_ref[...] + p_ref[0] + p_ref[1]
        z = jnp.dot(z, w_ref[...], preferred_element_type=jnp.float32)
        z = z + b_ref[...]
        o_ref[...] = _leaky(_leaky(z))

    blk = 1024
    return pl.pallas_call(
        body,
        grid=(_RPAD // blk,),
        in_specs=[
            pl.BlockSpec((blk, _F), lambda i: (i, 0)),
            pl.BlockSpec((_NC, blk, _F), lambda i: (0, i, 0)),
            pl.BlockSpec((_F, _H), lambda i: (0, 0)),
            pl.BlockSpec((1, _H), lambda i: (0, 0)),
        ],
        out_specs=pl.BlockSpec((blk, _H), lambda i: (i, 0)),
        out_shape=jax.ShapeDtypeStruct((_RPAD, _H), jnp.float32),
    )(x, parts, W0, b0)


def _tc_layer2(h1, parts, W1, b1):
    def body(h_ref, p_ref, w_ref, b_ref, o_ref):
        h = h_ref[...]
        za = h[:, :_F] + p_ref[0]
        zb = h[:, _F:] + p_ref[1]
        w = w_ref[...]
        z = jnp.dot(za, w[:_F, :], preferred_element_type=jnp.float32)
        z = z + jnp.dot(zb, w[_F:, :], preferred_element_type=jnp.float32)
        z = z + b_ref[...]
        o_ref[...] = _leaky(_leaky(z))

    blk = 1024
    return pl.pallas_call(
        body,
        grid=(_RPAD // blk,),
        in_specs=[
            pl.BlockSpec((blk, _H), lambda i: (i, 0)),
            pl.BlockSpec((_NC, blk, _F), lambda i: (0, i, 0)),
            pl.BlockSpec((_H, _H), lambda i: (0, 0)),
            pl.BlockSpec((1, _H), lambda i: (0, 0)),
        ],
        out_specs=pl.BlockSpec((blk, _H), lambda i: (i, 0)),
        out_shape=jax.ShapeDtypeStruct((_RPAD, _H), jnp.float32),
    )(h1, parts, W1, b1)


def _tc_pool_combine(sums, maxs, Wp, bp):
    def body(s_ref, m_ref, w_ref, b_ref, o_ref):
        sall = jnp.sum(s_ref[...], axis=0)
        ssum = sall[:, :_H]
        cnt = sall[:, _H]
        mx = jnp.max(m_ref[...], axis=0)
        mean = ssum / jnp.maximum(cnt, 1.0)[:, None]
        mx = jnp.where(cnt[:, None] > 0, mx, 0.0)
        g = jnp.concatenate([mean, ssum, mx], axis=-1)
        z = jnp.dot(g, w_ref[...], preferred_element_type=jnp.float32)
        o_ref[...] = _leaky(z + b_ref[...])

    return pl.pallas_call(
        body,
        grid=(1,),
        in_specs=[
            pl.BlockSpec((_NW, _G, _HS), lambda i: (0, 0, 0)),
            pl.BlockSpec((_NW, _G, _H), lambda i: (0, 0, 0)),
            pl.BlockSpec((3 * _H, _OUT), lambda i: (0, 0)),
            pl.BlockSpec((1, _OUT), lambda i: (0, 0)),
        ],
        out_specs=pl.BlockSpec((_G, _OUT), lambda i: (0, 0)),
        out_shape=jax.ShapeDtypeStruct((_G, _OUT), jnp.float32),
    )(sums, maxs, Wp, bp)


def kernel(x, edge_index, batch, W0, b0, W1, b1, Wp, bp):
    src = edge_index[0].astype(jnp.int32)
    dst = edge_index[1].astype(jnp.int32)

    pad1 = _NW * _K1 * _CH - _E
    src1 = jnp.concatenate([src, jnp.zeros((pad1,), jnp.int32)]).reshape(-1, _CH)
    dst1 = jnp.concatenate([dst, jnp.full((pad1,), _TRASH, jnp.int32)]).reshape(-1, _CH)
    pad2 = _NS * _K2 * _CH - _E
    src2 = jnp.concatenate([2 * src, jnp.zeros((pad2,), jnp.int32)]).reshape(-1, _CH)
    dst2 = jnp.concatenate([dst, jnp.full((pad2,), _TRASH, jnp.int32)]).reshape(-1, _CH)

    agg1 = _make_agg(_K1, add_core=False)(x, src1, dst1)
    h1 = _tc_layer1(x, agg1, W0, b0.reshape(1, _H))

    table2 = h1.reshape(2 * _RPAD, _F)
    agg2 = _make_agg(_K2, add_core=True)(table2, src2, dst2)
    h2 = _tc_layer2(h1, agg2, W1, b1.reshape(1, _H))

    batchp = jnp.concatenate(
        [batch.astype(jnp.int32), jnp.full((_RPAD - _N,), _G, jnp.int32)])
    sums, maxs = _pool(h2.reshape(-1), batchp)

    return _tc_pool_combine(sums.reshape(_NW, _G, _HS),
                            maxs.reshape(_NW, _G, _H),
                            Wp, bp.reshape(1, _OUT))


# speedup vs baseline: 1.2859x; 1.0023x over previous
import functools

import jax
import jax.numpy as jnp
from jax import lax
from jax.experimental import pallas as pl
from jax.experimental.pallas import tpu as pltpu
from jax.experimental.pallas import tpu_sc as plsc

_N = 10000
_E = 320000
_F = 128
_H = 256
_OUT = 128
_G = 128
_NEG = 0.01

_NC = 2
_NS = 16
_NW = _NC * _NS
_CH = 128
_IB = 40
_K1 = 79
_K2 = 157
_NP = 10112
_ZPW = _NP // _NS
_RPAD = 10240
_RPW = _RPAD // _NW
_TRASH = _N


def _leaky(x):
    return jnp.where(x >= 0, x, _NEG * x)


def _sc_mesh():
    return plsc.VectorSubcoreMesh(core_axis_name="c", subcore_axis_name="s")


def _make_agg(K, add_core):
    @functools.partial(
        pl.kernel,
        out_type=jax.ShapeDtypeStruct((_NC, _NP, _CH), jnp.float32),
        mesh=_sc_mesh(),
        scratch_types=[
            pltpu.VMEM((_CH,), jnp.int32),
            pltpu.VMEM((_CH,), jnp.int32),
            pltpu.VMEM((_CH, _CH), jnp.float32),
            pltpu.VMEM_SHARED((_NP, _CH), jnp.float32),
            pltpu.SemaphoreType.DMA,
        ],
    )
    def agg(table, srcm, dstm, out, idx_v, dst_v, rows_v, acc, sem):
        c = lax.axis_index("c")
        s = lax.axis_index("s")

        def zrow(i, carry):
            for k in range(_CH // 16):
                rows_v[i, pl.ds(k * 16, 16)] = jnp.zeros((16,), jnp.float32)
            return carry

        lax.fori_loop(0, _CH, zrow, 0)
        zbase = s * _ZPW
        off = 0
        while off < _ZPW:
            ch = min(_CH, _ZPW - off)
            pltpu.sync_copy(rows_v.at[pl.ds(0, ch)],
                            acc.at[pl.ds(zbase + off, ch)])
            off += ch
        plsc.subcore_barrier()

        if add_core:
            rbase = s * K
        else:
            rbase = (s * _NC + c) * K

        cvec = jnp.full((16,), c, jnp.int32)

        def body(j, carry):
            row = rbase + j
            pltpu.sync_copy(srcm.at[row], idx_v)
            if add_core:
                for k in range(_CH // 16):
                    idx_v[pl.ds(k * 16, 16)] = idx_v[pl.ds(k * 16, 16)] + cvec
            pltpu.async_copy(table.at[idx_v], rows_v, sem).wait()
            pltpu.sync_copy(dstm.at[row], dst_v)
            pltpu.sync_copy(rows_v, acc.at[dst_v], add=True)
            return carry

        lax.fori_loop(0, K, body, 0)
        plsc.subcore_barrier()
        pltpu.sync_copy(acc.at[pl.ds(zbase, _ZPW)],
                        out.at[c, pl.ds(zbase, _ZPW)])

    return agg


_HS = _H + 16


@functools.partial(
    pl.kernel,
    out_type=(
        jax.ShapeDtypeStruct((_NW, _G * _HS), jnp.float32),
        jax.ShapeDtypeStruct((_NW, _G * _H), jnp.float32),
    ),
    mesh=_sc_mesh(),
    scratch_types=[
        pltpu.VMEM((_RPW + 16,), jnp.int32),
        pltpu.VMEM((64 * _H,), jnp.float32),
        pltpu.VMEM(((_G + 1) * _HS,), jnp.float32),
        pltpu.VMEM(((_G + 1) * _H,), jnp.float32),
    ],
)
def _pool(h2f, batchp, sums, maxs, batch_v, buf, sacc, macc):
    c = lax.axis_index("c")
    s = lax.axis_index("s")
    wid = s * _NC + c
    base = wid * _RPW

    zero16 = jnp.zeros((16,), jnp.float32)
    ninf16 = jnp.full((16,), -3.4e38, jnp.float32)
    one16 = jnp.full((16,), 1.0, jnp.float32)

    def inits(i, carry):
        for u in range(4):
            sacc[pl.ds(i * 64 + u * 16, 16)] = zero16
        return carry

    def initm(i, carry):
        for u in range(4):
            macc[pl.ds(i * 64 + u * 16, 16)] = ninf16
        return carry

    lax.fori_loop(0, (_G + 1) * _HS // 64, inits, 0)
    for r in range((_G + 1) * _HS // 64 * 64, (_G + 1) * _HS, 16):
        sacc[pl.ds(r, 16)] = zero16
    lax.fori_loop(0, (_G + 1) * _H // 64, initm, 0)

    pltpu.sync_copy(batchp.at[pl.ds(base, _RPW)], batch_v.at[pl.ds(0, _RPW)])

    def chunk(t, carry):
        pltpu.sync_copy(h2f.at[pl.ds((base + t * 64) * _H, 64 * _H)], buf)

        def row(i, carry2):
            r = t * 64 + i
            g = batch_v[pl.ds(r, 16)][0]
            soff = g * _HS
            moff = g * _H
            for k in range(_H // 16):
                v = buf[pl.ds(i * _H + k * 16, 16)]
                sacc[pl.ds(soff + k * 16, 16)] = sacc[pl.ds(soff + k * 16, 16)] + v
                m = macc[pl.ds(moff + k * 16, 16)]
                macc[pl.ds(moff + k * 16, 16)] = jnp.maximum(m, v)
            sacc[pl.ds(soff + _H, 16)] = sacc[pl.ds(soff + _H, 16)] + one16
            return carry2

        lax.fori_loop(0, 64, row, 0)
        return carry

    lax.fori_loop(0, _RPW // 64, chunk, 0)

    pltpu.sync_copy(sacc.at[pl.ds(0, _G * _HS)], sums.at[wid])
    pltpu.sync_copy(macc.at[pl.ds(0, _G * _H)], maxs.at[wid])


def _tc_layer1(x, parts, W0, b0):
    def body(x_ref, p_ref, w_ref, b_ref, o_ref):
        z = x_ref[...] + p_ref[0] + p_ref[1]
        z = jnp.dot(z, w_ref[...], preferred_element_type=jnp.float32)
        z = z + b_ref[...]
        o_ref[...] = _leaky(_leaky(z))

    blk = 1024
    return pl.pallas_call(
        body,
        grid=(_RPAD // blk,),
        in_specs=[
            pl.BlockSpec((blk, _F), lambda i: (i, 0)),
            pl.BlockSpec((_NC, blk, _F), lambda i: (0, i, 0)),
            pl.BlockSpec((_F, _H), lambda i: (0, 0)),
            pl.BlockSpec((1, _H), lambda i: (0, 0)),
        ],
        out_specs=pl.BlockSpec((blk, _H), lambda i: (i, 0)),
        out_shape=jax.ShapeDtypeStruct((_RPAD, _H), jnp.float32),
    )(x, parts, W0, b0)


def _tc_layer2(h1, parts, W1, b1):
    def body(h_ref, p_ref, w_ref, b_ref, o_ref):
        h = h_ref[...]
        za = h[:, :_F] + p_ref[0]
        zb = h[:, _F:] + p_ref[1]
        w = w_ref[...]
        z = jnp.dot(za, w[:_F, :], preferred_element_type=jnp.float32)
        z = z + jnp.dot(zb, w[_F:, :], preferred_element_type=jnp.float32)
        z = z + b_ref[...]
        o_ref[...] = _leaky(_leaky(z))

    blk = 1024
    return pl.pallas_call(
        body,
        grid=(_RPAD // blk,),
        in_specs=[
            pl.BlockSpec((blk, _H), lambda i: (i, 0)),
            pl.BlockSpec((_NC, blk, _F), lambda i: (0, i, 0)),
            pl.BlockSpec((_H, _H), lambda i: (0, 0)),
            pl.BlockSpec((1, _H), lambda i: (0, 0)),
        ],
        out_specs=pl.BlockSpec((blk, _H), lambda i: (i, 0)),
        out_shape=jax.ShapeDtypeStruct((_RPAD, _H), jnp.float32),
    )(h1, parts, W1, b1)


def _tc_pool_combine(sums, maxs, Wp, bp):
    def body(s_ref, m_ref, w_ref, b_ref, o_ref):
        sall = jnp.sum(s_ref[...], axis=0)
        ssum = sall[:, :_H]
        cnt = sall[:, _H]
        mx = jnp.max(m_ref[...], axis=0)
        mean = ssum / jnp.maximum(cnt, 1.0)[:, None]
        mx = jnp.where(cnt[:, None] > 0, mx, 0.0)
        g = jnp.concatenate([mean, ssum, mx], axis=-1)
        z = jnp.dot(g, w_ref[...], preferred_element_type=jnp.float32)
        o_ref[...] = _leaky(z + b_ref[...])

    return pl.pallas_call(
        body,
        grid=(1,),
        in_specs=[
            pl.BlockSpec((_NW, _G, _HS), lambda i: (0, 0, 0)),
            pl.BlockSpec((_NW, _G, _H), lambda i: (0, 0, 0)),
            pl.BlockSpec((3 * _H, _OUT), lambda i: (0, 0)),
            pl.BlockSpec((1, _OUT), lambda i: (0, 0)),
        ],
        out_specs=pl.BlockSpec((_G, _OUT), lambda i: (0, 0)),
        out_shape=jax.ShapeDtypeStruct((_G, _OUT), jnp.float32),
    )(sums, maxs, Wp, bp)


def kernel(x, edge_index, batch, W0, b0, W1, b1, Wp, bp):
    src = edge_index[0].astype(jnp.int32)
    dst = edge_index[1].astype(jnp.int32)

    pad1 = _NW * _K1 * _CH - _E
    src1 = jnp.concatenate([src, jnp.zeros((pad1,), jnp.int32)]).reshape(-1, _CH)
    dst1 = jnp.concatenate([dst, jnp.full((pad1,), _TRASH, jnp.int32)]).reshape(-1, _CH)
    pad2 = _NS * _K2 * _CH - _E
    src2 = jnp.concatenate([2 * src, jnp.zeros((pad2,), jnp.int32)]).reshape(-1, _CH)
    dst2 = jnp.concatenate([dst, jnp.full((pad2,), _TRASH, jnp.int32)]).reshape(-1, _CH)

    agg1 = _make_agg(_K1, add_core=False)(x, src1, dst1)
    h1 = _tc_layer1(x, agg1, W0, b0.reshape(1, _H))

    table2 = h1.reshape(2 * _RPAD, _F)
    agg2 = _make_agg(_K2, add_core=True)(table2, src2, dst2)
    h2 = _tc_layer2(h1, agg2, W1, b1.reshape(1, _H))

    batchp = jnp.concatenate(
        [batch.astype(jnp.int32), jnp.full((_RPAD - _N,), _G, jnp.int32)])
    sums, maxs = _pool(h2.reshape(-1), batchp)

    return _tc_pool_combine(sums.reshape(_NW, _G, _HS),
                            maxs.reshape(_NW, _G, _H),
                            Wp, bp.reshape(1, _OUT))
